# baseline (speedup 1.0000x reference)
import functools

import jax
import jax.numpy as jnp
from jax import lax
from jax.experimental import pallas as pl
from jax.experimental.pallas import tpu as pltpu
from jax.experimental.pallas import tpu_sc as plsc

N = 10000
E = 160000
IN = 16
OUT = 16
EF = 2
HID = IN * OUT

_NW = 32


_GPW = E // _NW


def _run_gather(v, src):
    mesh = plsc.VectorSubcoreMesh(core_axis_name="c", subcore_axis_name="s")

    @functools.partial(
        pl.kernel,
        mesh=mesh,
        out_type=jax.ShapeDtypeStruct((E, IN), jnp.float32),
        scratch_types=[
            pltpu.VMEM((_GPW,), jnp.int32),
            pltpu.VMEM((_GPW, IN), jnp.float32),
            pltpu.SemaphoreType.DMA,
        ],
        compiler_params=pltpu.CompilerParams(use_tc_tiling_on_sc=False),
    )
    def gather_k(v_hbm, src_hbm, out_hbm, idx_v, rows_v, sem):
        wid = lax.axis_index("s") * 2 + lax.axis_index("c")
        base = wid * _GPW
        pltpu.sync_copy(src_hbm.at[pl.ds(base, _GPW)], idx_v)
        pltpu.async_copy(v_hbm.at[idx_v], rows_v, sem).wait()
        pltpu.sync_copy(rows_v, out_hbm.at[pl.ds(base, _GPW)])

    return gather_k(v, src)


_STATS_BW = 6400
_STATS_STEPS = E // _STATS_BW
_MSG_B = 3200
_MSG_STEPS = E // _MSG_B
_TOT_STEPS = _STATS_STEPS + _MSG_STEPS


def _mega_body(attr_t_ref, w_t_ref, b_ref, gamma_ref, beta_ref, v_ref,
               wroot_ref, bconv_ref, xj_ref, attr_tm_ref, r_ref, s_ref,
               msg_ref, root_ref, acc_ref, cd_ref):
    step = pl.program_id(0)

    @pl.when(step == 0)
    def _init():
        acc_ref[...] = jnp.zeros_like(acc_ref)

    @pl.when(step < _STATS_STEPS)
    def _stats():
        r0 = attr_t_ref[0:1, :]
        r1 = attr_t_ref[1:2, :]
        acc_ref[0:1, :] += r0
        acc_ref[1:2, :] += r1
        acc_ref[2:3, :] += r0 * r0
        acc_ref[3:4, :] += r0 * r1
        acc_ref[4:5, :] += r1 * r1

    @pl.when(step == _STATS_STEPS - 1)
    def _fold():
        inv_e = 1.0 / E
        m0 = jnp.sum(acc_ref[0:1, :]) * inv_e
        m1 = jnp.sum(acc_ref[1:2, :]) * inv_e
        c00 = jnp.sum(acc_ref[2:3, :]) * inv_e - m0 * m0
        c01 = jnp.sum(acc_ref[3:4, :]) * inv_e - m0 * m1
        c11 = jnp.sum(acc_ref[4:5, :]) * inv_e - m1 * m1
        w0 = w_t_ref[0:1, :]
        w1 = w_t_ref[1:2, :]
        mu = w0 * m0 + w1 * m1 + b_ref[...]
        var = w0 * w0 * c00 + 2.0 * (w0 * w1) * c01 + w1 * w1 * c11
        inv = gamma_ref[...] * lax.rsqrt(var + 1e-5)
        cd_ref[0:1, :] = w0 * inv
        cd_ref[1:2, :] = w1 * inv
        cd_ref[2:3, :] = (b_ref[...] - mu) * inv + beta_ref[...]
        root_ref[...] = (
            jnp.dot(v_ref[...], wroot_ref[...],
                    preferred_element_type=jnp.float32)
            + bconv_ref[...]
        )

    @pl.when(step >= _STATS_STEPS)
    def _msg():
        cmat = cd_ref[0:2, :].astype(jnp.bfloat16)
        d = cd_ref[2:3, :]
        pre = lax.dot_general(
            attr_tm_ref[...].astype(jnp.bfloat16), cmat,
            (((0,), (0,)), ((), ())),
            preferred_element_type=jnp.float32,
        )
        h = jnp.tanh(pre + d)
        xr = jnp.dot(xj_ref[...], r_ref[...],
                     preferred_element_type=jnp.float32)
        msg = jnp.dot((xr * h).astype(jnp.bfloat16),
                      s_ref[...].astype(jnp.bfloat16),
                      preferred_element_type=jnp.float32)
        ones_col = (
            lax.broadcasted_iota(jnp.int32, (_MSG_B, 32), 1) == IN
        ).astype(jnp.float32)
        msg_ref[...] = msg + ones_col


def _run_mega(attr_t, w_t, b_enet, gamma, beta, v, w_root, b_conv, xj,
              rmat, smat):
    cmap = lambda i: (0, 0)
    smap = lambda i: (0, jnp.minimum(i, _STATS_STEPS - 1))
    mmap = lambda i: (jnp.maximum(i - _STATS_STEPS, 0), 0)
    mmap2 = lambda i: (0, jnp.maximum(i - _STATS_STEPS, 0))
    return pl.pallas_call(
        _mega_body,
        grid=(_TOT_STEPS,),
        in_specs=[
            pl.BlockSpec((2, _STATS_BW), smap),
            pl.BlockSpec((2, HID), cmap),
            pl.BlockSpec((1, HID), cmap),
            pl.BlockSpec((1, HID), cmap),
            pl.BlockSpec((1, HID), cmap),
            pl.BlockSpec((N, IN), cmap),
            pl.BlockSpec((IN, OUT), cmap),
            pl.BlockSpec((1, OUT), cmap),
            pl.BlockSpec((_MSG_B, IN), mmap),
            pl.BlockSpec((2, _MSG_B), mmap2),
            pl.BlockSpec((IN, HID), cmap),
            pl.BlockSpec((HID, 32), cmap),
        ],
        out_specs=[
            pl.BlockSpec((_MSG_B, 32), mmap),
            pl.BlockSpec((N, OUT), cmap),
        ],
        out_shape=[
            jax.ShapeDtypeStruct((E, 32), jnp.float32),
            jax.ShapeDtypeStruct((N, OUT), jnp.float32),
        ],
        scratch_shapes=[
            pltpu.VMEM((8, _STATS_BW), jnp.float32),
            pltpu.VMEM((8, HID), jnp.float32),
        ],
    )(attr_t, w_t, b_enet, gamma, beta, v, w_root, b_conv, xj, attr_t,
      rmat, smat)


_SPW = 1000
_SCHUNKS = _GPW // _SPW
_NPT = N // 16


def _run_scatter(msg, dst, zeros):
    mesh = plsc.VectorSubcoreMesh(core_axis_name="c", subcore_axis_name="s")

    @functools.partial(
        pl.kernel,
        mesh=mesh,
        out_type=jax.ShapeDtypeStruct((2, N, 32), jnp.float32),
        scratch_types=[
            pltpu.VMEM((_SPW,), jnp.int32),
            pltpu.VMEM((_SPW, 32), jnp.float32),
            pltpu.VMEM_SHARED((N, 32), jnp.float32),
        ],
        compiler_params=pltpu.CompilerParams(use_tc_tiling_on_sc=False),
    )
    def scatter_k(msg_hbm, dst_hbm, zeros_hbm, out_hbm, idx_v, val_v, shared):
        cid = lax.axis_index("c")
        sid = lax.axis_index("s")
        pltpu.sync_copy(
            zeros_hbm.at[pl.ds(sid * _NPT, _NPT)],
            shared.at[pl.ds(sid * _NPT, _NPT)],
        )
        plsc.subcore_barrier()
        wid = sid * 2 + cid
        for c in range(_SCHUNKS):
            base = wid * _GPW + c * _SPW
            pltpu.sync_copy(dst_hbm.at[pl.ds(base, _SPW)], idx_v)
            pltpu.sync_copy(msg_hbm.at[pl.ds(base, _SPW)], val_v)
            pltpu.sync_copy(val_v, shared.at[idx_v], add=True)
        plsc.subcore_barrier()
        pltpu.sync_copy(
            shared.at[pl.ds(sid * _NPT, _NPT)],
            out_hbm.at[cid, pl.ds(sid * _NPT, _NPT)],
        )

    return scatter_k(msg, dst, zeros)


def _final_body(p0_ref, p1_ref, root_ref, out_ref):
    s = p0_ref[:, 0:IN] + p1_ref[:, 0:IN]
    cnt = p0_ref[:, IN : IN + 1] + p1_ref[:, IN : IN + 1]
    o = s / jnp.maximum(cnt, 1.0) + root_ref[...]
    out_ref[...] = jnp.where(o >= 0, o, 0.01 * o)


def _run_final(p0, p1, root):
    return pl.pallas_call(
        _final_body,
        grid=(1,),
        in_specs=[
            pl.BlockSpec((N, 32), lambda i: (0, 0)),
            pl.BlockSpec((N, 32), lambda i: (0, 0)),
            pl.BlockSpec((N, OUT), lambda i: (0, 0)),
        ],
        out_specs=pl.BlockSpec((N, OUT), lambda i: (0, 0)),
        out_shape=jax.ShapeDtypeStruct((N, OUT), jnp.float32),
    )(p0, p1, root)


@jax.jit
def _kernel_impl(v, edge_index, edge_attr, W_enet, b_enet, bn_gamma, bn_beta,
                 W_root, b_conv):
    src = edge_index[0]
    dst = edge_index[1]
    xj = _run_gather(v, src)
    jj = jnp.arange(HID, dtype=jnp.int32)
    rmat = (jj[None, :] // IN == jnp.arange(IN, dtype=jnp.int32)[:, None]).astype(
        jnp.float32
    )
    oo = jnp.arange(32, dtype=jnp.int32)
    smat = ((jj[:, None] % IN == oo[None, :]) & (oo[None, :] < IN)).astype(
        jnp.float32
    )
    msg, root = _run_mega(
        edge_attr.T,
        W_enet.T,
        b_enet.reshape(1, HID),
        bn_gamma.reshape(1, HID),
        bn_beta.reshape(1, HID),
        v,
        W_root,
        b_conv.reshape(1, OUT),
        xj,
        rmat,
        smat,
    )
    partials = _run_scatter(msg, dst, jnp.zeros((N, 32), jnp.float32))
    return _run_final(partials[0], partials[1], root)


def kernel(v, edge_index, edge_attr, W_enet, b_enet, bn_gamma, bn_beta,
           W_root, b_conv):
    return _kernel_impl(v, edge_index, edge_attr, W_enet, b_enet, bn_gamma,
                        bn_beta, W_root, b_conv)


# speedup vs baseline: 1.0557x; 1.0557x over previous
import functools

import jax
import jax.numpy as jnp
from jax import lax
from jax.experimental import pallas as pl
from jax.experimental.pallas import tpu as pltpu
from jax.experimental.pallas import tpu_sc as plsc

N = 10000
E = 160000
IN = 16
OUT = 16
EF = 2
HID = IN * OUT

_NW = 32


_GPW = E // _NW


def _run_gather(v, src):
    mesh = plsc.VectorSubcoreMesh(core_axis_name="c", subcore_axis_name="s")

    @functools.partial(
        pl.kernel,
        mesh=mesh,
        out_type=jax.ShapeDtypeStruct((E, IN), jnp.float32),
        scratch_types=[
            pltpu.VMEM((_GPW,), jnp.int32),
            pltpu.VMEM((_GPW, IN), jnp.float32),
            pltpu.SemaphoreType.DMA,
        ],
        compiler_params=pltpu.CompilerParams(use_tc_tiling_on_sc=False),
    )
    def gather_k(v_hbm, src_hbm, out_hbm, idx_v, rows_v, sem):
        wid = lax.axis_index("s") * 2 + lax.axis_index("c")
        base = wid * _GPW
        pltpu.sync_copy(src_hbm.at[pl.ds(base, _GPW)], idx_v)
        pltpu.async_copy(v_hbm.at[idx_v], rows_v, sem).wait()
        pltpu.sync_copy(rows_v, out_hbm.at[pl.ds(base, _GPW)])

    return gather_k(v, src)


_STATS_BW = 6400
_STATS_STEPS = E // _STATS_BW
_MSG_B = 6400
_MSG_STEPS = E // _MSG_B
_TOT_STEPS = _STATS_STEPS + _MSG_STEPS


def _mega_body(attr_t_ref, w_t_ref, b_ref, gamma_ref, beta_ref, v_ref,
               wroot_ref, bconv_ref, xj_ref, attr_tm_ref, r_ref, s_ref,
               msg_ref, root_ref, acc_ref, cd_ref):
    step = pl.program_id(0)

    @pl.when(step == 0)
    def _init():
        acc_ref[...] = jnp.zeros_like(acc_ref)

    @pl.when(step < _STATS_STEPS)
    def _stats():
        r0 = attr_t_ref[0:1, :]
        r1 = attr_t_ref[1:2, :]
        acc_ref[0:1, :] += r0
        acc_ref[1:2, :] += r1
        acc_ref[2:3, :] += r0 * r0
        acc_ref[3:4, :] += r0 * r1
        acc_ref[4:5, :] += r1 * r1

    @pl.when(step == _STATS_STEPS - 1)
    def _fold():
        inv_e = 1.0 / E
        m0 = jnp.sum(acc_ref[0:1, :]) * inv_e
        m1 = jnp.sum(acc_ref[1:2, :]) * inv_e
        c00 = jnp.sum(acc_ref[2:3, :]) * inv_e - m0 * m0
        c01 = jnp.sum(acc_ref[3:4, :]) * inv_e - m0 * m1
        c11 = jnp.sum(acc_ref[4:5, :]) * inv_e - m1 * m1
        w0 = w_t_ref[0:1, :]
        w1 = w_t_ref[1:2, :]
        mu = w0 * m0 + w1 * m1 + b_ref[...]
        var = w0 * w0 * c00 + 2.0 * (w0 * w1) * c01 + w1 * w1 * c11
        inv = gamma_ref[...] * lax.rsqrt(var + 1e-5)
        cd_ref[0:1, :] = w0 * inv
        cd_ref[1:2, :] = w1 * inv
        cd_ref[2:3, :] = (b_ref[...] - mu) * inv + beta_ref[...]
        root_ref[...] = (
            jnp.dot(v_ref[...], wroot_ref[...],
                    preferred_element_type=jnp.float32)
            + bconv_ref[...]
        )

    @pl.when(step >= _STATS_STEPS)
    def _msg():
        cmat = cd_ref[0:2, :]
        d = cd_ref[2:3, :]
        pre = lax.dot_general(
            attr_tm_ref[...], cmat, (((0,), (0,)), ((), ())),
            preferred_element_type=jnp.float32,
        )
        h = jnp.tanh(pre + d)
        xr = jnp.dot(xj_ref[...], r_ref[...],
                     preferred_element_type=jnp.float32)
        msg = jnp.dot(xr * h, s_ref[...], preferred_element_type=jnp.float32)
        ones_col = (
            lax.broadcasted_iota(jnp.int32, (_MSG_B, 32), 1) == IN
        ).astype(jnp.float32)
        msg_ref[...] = msg + ones_col


def _run_mega(attr_t, w_t, b_enet, gamma, beta, v, w_root, b_conv, xj,
              rmat, smat):
    cmap = lambda i: (0, 0)
    smap = lambda i: (0, jnp.minimum(i, _STATS_STEPS - 1))
    mmap = lambda i: (jnp.maximum(i - _STATS_STEPS, 0), 0)
    mmap2 = lambda i: (0, jnp.maximum(i - _STATS_STEPS, 0))
    return pl.pallas_call(
        _mega_body,
        grid=(_TOT_STEPS,),
        in_specs=[
            pl.BlockSpec((2, _STATS_BW), smap),
            pl.BlockSpec((2, HID), cmap),
            pl.BlockSpec((1, HID), cmap),
            pl.BlockSpec((1, HID), cmap),
            pl.BlockSpec((1, HID), cmap),
            pl.BlockSpec((N, IN), cmap),
            pl.BlockSpec((IN, OUT), cmap),
            pl.BlockSpec((1, OUT), cmap),
            pl.BlockSpec((_MSG_B, IN), mmap),
            pl.BlockSpec((2, _MSG_B), mmap2),
            pl.BlockSpec((IN, HID), cmap),
            pl.BlockSpec((HID, 32), cmap),
        ],
        out_specs=[
            pl.BlockSpec((_MSG_B, 32), mmap),
            pl.BlockSpec((N, OUT), cmap),
        ],
        out_shape=[
            jax.ShapeDtypeStruct((E, 32), jnp.float32),
            jax.ShapeDtypeStruct((N, OUT), jnp.float32),
        ],
        scratch_shapes=[
            pltpu.VMEM((8, _STATS_BW), jnp.float32),
            pltpu.VMEM((8, HID), jnp.float32),
        ],
    )(attr_t, w_t, b_enet, gamma, beta, v, w_root, b_conv, xj, attr_t,
      rmat, smat)


_SPW = 1000
_SCHUNKS = _GPW // _SPW
_NPT = N // 16


def _run_scatter(msg, dst, zeros):
    mesh = plsc.VectorSubcoreMesh(core_axis_name="c", subcore_axis_name="s")

    @functools.partial(
        pl.kernel,
        mesh=mesh,
        out_type=jax.ShapeDtypeStruct((2, N, 32), jnp.float32),
        scratch_types=[
            pltpu.VMEM((_SPW,), jnp.int32),
            pltpu.VMEM((_SPW, 32), jnp.float32),
            pltpu.VMEM_SHARED((N, 32), jnp.float32),
        ],
        compiler_params=pltpu.CompilerParams(use_tc_tiling_on_sc=False),
    )
    def scatter_k(msg_hbm, dst_hbm, zeros_hbm, out_hbm, idx_v, val_v, shared):
        cid = lax.axis_index("c")
        sid = lax.axis_index("s")
        pltpu.sync_copy(
            zeros_hbm.at[pl.ds(sid * _NPT, _NPT)],
            shared.at[pl.ds(sid * _NPT, _NPT)],
        )
        plsc.subcore_barrier()
        wid = sid * 2 + cid
        for c in range(_SCHUNKS):
            base = wid * _GPW + c * _SPW
            pltpu.sync_copy(dst_hbm.at[pl.ds(base, _SPW)], idx_v)
            pltpu.sync_copy(msg_hbm.at[pl.ds(base, _SPW)], val_v)
            pltpu.sync_copy(val_v, shared.at[idx_v], add=True)
        plsc.subcore_barrier()
        pltpu.sync_copy(
            shared.at[pl.ds(sid * _NPT, _NPT)],
            out_hbm.at[cid, pl.ds(sid * _NPT, _NPT)],
        )

    return scatter_k(msg, dst, zeros)


def _final_body(p0_ref, p1_ref, root_ref, out_ref):
    s = p0_ref[:, 0:IN] + p1_ref[:, 0:IN]
    cnt = p0_ref[:, IN : IN + 1] + p1_ref[:, IN : IN + 1]
    o = s / jnp.maximum(cnt, 1.0) + root_ref[...]
    out_ref[...] = jnp.where(o >= 0, o, 0.01 * o)


def _run_final(p0, p1, root):
    return pl.pallas_call(
        _final_body,
        grid=(1,),
        in_specs=[
            pl.BlockSpec((N, 32), lambda i: (0, 0)),
            pl.BlockSpec((N, 32), lambda i: (0, 0)),
            pl.BlockSpec((N, OUT), lambda i: (0, 0)),
        ],
        out_specs=pl.BlockSpec((N, OUT), lambda i: (0, 0)),
        out_shape=jax.ShapeDtypeStruct((N, OUT), jnp.float32),
    )(p0, p1, root)


@jax.jit
def _kernel_impl(v, edge_index, edge_attr, W_enet, b_enet, bn_gamma, bn_beta,
                 W_root, b_conv):
    src = edge_index[0]
    dst = edge_index[1]
    xj = _run_gather(v, src)
    jj = jnp.arange(HID, dtype=jnp.int32)
    rmat = (jj[None, :] // IN == jnp.arange(IN, dtype=jnp.int32)[:, None]).astype(
        jnp.float32
    )
    oo = jnp.arange(32, dtype=jnp.int32)
    smat = ((jj[:, None] % IN == oo[None, :]) & (oo[None, :] < IN)).astype(
        jnp.float32
    )
    msg, root = _run_mega(
        edge_attr.T,
        W_enet.T,
        b_enet.reshape(1, HID),
        bn_gamma.reshape(1, HID),
        bn_beta.reshape(1, HID),
        v,
        W_root,
        b_conv.reshape(1, OUT),
        xj,
        rmat,
        smat,
    )
    partials = _run_scatter(msg, dst, jnp.zeros((N, 32), jnp.float32))
    return _run_final(partials[0], partials[1], root)


def kernel(v, edge_index, edge_attr, W_enet, b_enet, bn_gamma, bn_beta,
           W_root, b_conv):
    return _kernel_impl(v, edge_index, edge_attr, W_enet, b_enet, bn_gamma,
                        bn_beta, W_root, b_conv)
